# baseline (speedup 1.0000x reference)
import jax
import jax.numpy as jnp
from jax import lax
from jax.experimental import pallas as pl
from jax.experimental.pallas import tpu as pltpu
from jax.experimental.pallas import tpu_sc as plsc

N = 10000
E = 160000
D = 256
BLK = 200
GRID = N // BLK
ABLK = 400
AGRID = N // ABLK
HALF = N // 2
NSUB = 16
NCORE = 2
CHUNK = E // NSUB
SEC = 2000
NSEC = CHUNK // SEC
SGROUPS = SEC // 16
G = 32
CAP = SEC + 2 * G
DLOC = HALF
ZROWS = 1000


def _phase_a(x_ref, lbl_ref, w_ref, att_ref, ox_ref, oxb_ref, s_ref):
    xb = x_ref[...]
    lbl = lbl_ref[0, 0, :]
    acc = xb
    for i in range(7):
        m = (lbl == (i + 1)).astype(jnp.float32)[:, None]
        acc = acc + m * jnp.dot(xb, w_ref[i], preferred_element_type=jnp.float32)
    ox_ref[...] = acc
    accb = acc.astype(jnp.bfloat16)
    oxb_ref[...] = accb.reshape(ABLK, D // 32, 2, 16).transpose(
        0, 1, 3, 2).reshape(ABLK, D)
    s_ref[0, 0, :] = jnp.dot(acc, att_ref[0], preferred_element_type=jnp.float32)
    s_ref[0, 1, :] = jnp.dot(acc, att_ref[1], preferred_element_type=jnp.float32)


def _run_phase_a(x, lbl3, w_id, att8):
    return pl.pallas_call(
        _phase_a,
        grid=(AGRID,),
        in_specs=[
            pl.BlockSpec((ABLK, D), lambda j: (j, 0)),
            pl.BlockSpec((1, 1, ABLK), lambda j: (j, 0, 0)),
            pl.BlockSpec((7, D, D), lambda j: (0, 0, 0)),
            pl.BlockSpec((8, D), lambda j: (0, 0)),
        ],
        out_specs=[
            pl.BlockSpec((ABLK, D), lambda j: (j, 0)),
            pl.BlockSpec((ABLK, D), lambda j: (j, 0)),
            pl.BlockSpec((1, 2, ABLK), lambda j: (j, 0, 0)),
        ],
        out_shape=[
            jax.ShapeDtypeStruct((N, D), jnp.float32),
            jax.ShapeDtypeStruct((N, D), jnp.bfloat16),
            jax.ShapeDtypeStruct((AGRID, 2, ABLK), jnp.float32),
        ],
    )(x, lbl3, w_id, att8)


def _phase_b_impl(src_hbm, dst_hbm, sdst_hbm, ssrc_hbm, oxb_hbm, zrows_hbm,
                  zden_hbm, num_hbm, den_hbm,
                  sdst_v, ssrc_v, rawsrc, rawdst, csrc, cdst, cw, rows_a,
                  rows_b, srows, sc_idx, den_loc, num_sh, sem_a, sem_b):
    core = lax.axis_index("c")
    sub = lax.axis_index("s")
    base_dst = core * HALF

    @pl.when(sub < HALF // ZROWS)
    def _zero_num():
        pltpu.sync_copy(zrows_hbm, num_sh.at[pl.ds(sub * ZROWS, ZROWS)])

    pltpu.sync_copy(zden_hbm, den_loc)

    pltpu.sync_copy(sdst_hbm.at[pl.ds(base_dst, HALF)], sdst_v)
    pltpu.sync_copy(ssrc_hbm, ssrc_v)
    plsc.subcore_barrier()

    def start_gather(b, rows_k, sem_k):
        pltpu.async_copy(oxb_hbm.at[csrc.at[pl.ds(b * G, G)]], rows_k, sem_k)

    def finish_block(b, rows_k, sem_k):
        pltpu.make_async_copy(oxb_hbm.at[pl.ds(0, G)], rows_k, sem_k).wait()
        off = b * G
        for k in range(G // 16):
            sc_idx[pl.ds(16 * k, 16)] = cdst[pl.ds(off + 16 * k, 16)]

        def scale_group(q, c2):
            w16 = cw[pl.ds(off + 16 * q, 16)]
            for j in range(16):
                r = 16 * q + j
                wv = jnp.full((16,), w16[j], jnp.float32)
                for c in range(D // 32):
                    ab = rows_k[r, pl.ds(32 * c, 32)]
                    lo, hi = plsc.unpack(ab, format=plsc.PackFormat.INTERLEAVED)
                    srows[r, pl.ds(32 * c, 16)] = lo * wv
                    srows[r, pl.ds(32 * c + 16, 16)] = hi * wv
            return c2

        lax.fori_loop(0, G // 16, scale_group, 0)
        pltpu.sync_copy(srows, num_sh.at[sc_idx], add=True)

    def do_section(sec, ptr):
        ebase = sub * CHUNK + sec * SEC
        pltpu.sync_copy(src_hbm.at[pl.ds(ebase, SEC)], rawsrc)
        pltpu.sync_copy(dst_hbm.at[pl.ds(ebase, SEC)], rawdst)

        def scan_group(g, p):
            src16 = rawsrc[pl.ds(g * 16, 16)]
            dst16 = rawdst[pl.ds(g * 16, 16)]
            local = dst16 - base_dst
            keep = ((local >= 0) & (local < HALF)) & (src16 != dst16)
            safe_local = jnp.where(keep, local, 0)
            a = (plsc.load_gather(sdst_v, [safe_local])
                 + plsc.load_gather(ssrc_v, [src16]))
            a = jnp.where(a > 0, a, 0.2 * a)
            w16 = jnp.exp(a)
            plsc.addupdate_scatter(den_loc, [safe_local], w16, mask=keep)
            csum = jnp.cumsum(keep.astype(jnp.int32))
            pos = p + csum - 1
            plsc.store_scatter(csrc, [pos], src16, mask=keep)
            plsc.store_scatter(cdst, [pos], safe_local, mask=keep)
            plsc.store_scatter(cw, [pos], w16, mask=keep)
            return p + jnp.max(csum)

        ptr = lax.fori_loop(0, SGROUPS, scan_group, ptr)

        nblk = ptr // G

        @pl.when(nblk > 0)
        def _prime_a():
            start_gather(0, rows_a, sem_a)

        @pl.when(nblk > 1)
        def _prime_b():
            start_gather(1, rows_b, sem_b)

        def outer(g2, c2):
            for k, (rk, sk) in enumerate(((rows_a, sem_a), (rows_b, sem_b))):
                b = g2 * 2 + k

                @pl.when(b < nblk)
                def _run():
                    finish_block(b, rk, sk)

                    @pl.when(b + 2 < nblk)
                    def _next():
                        start_gather(b + 2, rk, sk)
            return c2

        lax.fori_loop(0, (nblk + 1) // 2, outer, 0)
        rem = ptr - nblk * G
        for k in range(G // 16):
            sl = pl.ds(16 * k, 16)
            tmp_s = csrc[pl.ds(nblk * G + 16 * k, 16)]
            tmp_d = cdst[pl.ds(nblk * G + 16 * k, 16)]
            tmp_w = cw[pl.ds(nblk * G + 16 * k, 16)]
            csrc[sl] = tmp_s
            cdst[sl] = tmp_d
            cw[sl] = tmp_w
        return rem

    rem = lax.fori_loop(0, NSEC, do_section, jnp.int32(0))

    lane = lax.iota(jnp.int32, 16)
    for k in range(G // 16):
        pos = rem + lane + 16 * k
        plsc.store_scatter(csrc, [pos], jnp.zeros((16,), jnp.int32))
        plsc.store_scatter(cdst, [pos], jnp.zeros((16,), jnp.int32))
        plsc.store_scatter(cw, [pos], jnp.zeros((16,), jnp.float32))
    start_gather(0, rows_a, sem_a)
    finish_block(0, rows_a, sem_a)

    plsc.subcore_barrier()
    pltpu.sync_copy(den_loc, den_hbm.at[core, sub])

    @pl.when(sub < HALF // ZROWS)
    def _copy_out():
        row0 = sub * ZROWS
        pltpu.sync_copy(num_sh.at[pl.ds(row0, ZROWS)],
                        num_hbm.at[pl.ds(base_dst + row0, ZROWS)])


def _run_phase_b(src, dst, sdst, ssrc, oxb, zrows, zden):
    mesh = plsc.VectorSubcoreMesh(core_axis_name="c", subcore_axis_name="s")
    kern = pl.kernel(
        _phase_b_impl,
        mesh=mesh,
        compiler_params=pltpu.CompilerParams(
            use_tc_tiling_on_sc=False, needs_layout_passes=False),
        out_type=[
            jax.ShapeDtypeStruct((N, D), jnp.float32),
            jax.ShapeDtypeStruct((NCORE, NSUB, DLOC), jnp.float32),
        ],
        scratch_types=[
            pltpu.VMEM((HALF,), jnp.float32),
            pltpu.VMEM((N,), jnp.float32),
            pltpu.VMEM((SEC,), jnp.int32),
            pltpu.VMEM((SEC,), jnp.int32),
            pltpu.VMEM((CAP,), jnp.int32),
            pltpu.VMEM((CAP,), jnp.int32),
            pltpu.VMEM((CAP,), jnp.float32),
            pltpu.VMEM((G, D), jnp.bfloat16),
            pltpu.VMEM((G, D), jnp.bfloat16),
            pltpu.VMEM((G, D), jnp.float32),
            pltpu.VMEM((G,), jnp.int32),
            pltpu.VMEM((DLOC,), jnp.float32),
            pltpu.VMEM_SHARED((HALF, D), jnp.float32),
            pltpu.SemaphoreType.DMA,
            pltpu.SemaphoreType.DMA,
        ],
    )
    return kern(src, dst, sdst, ssrc, oxb, zrows, zden)


def _phase_d(num_ref, den_ref, ox_ref, s_ref, out_ref):
    den = jnp.sum(den_ref[0, 0], axis=0)
    a = s_ref[0, 0, :] + s_ref[0, 1, :]
    a = jnp.where(a > 0, a, 0.2 * a)
    l = jnp.exp(a)
    oxb = ox_ref[...]
    out_ref[...] = ((num_ref[...] + l[:, None] * oxb)
                    / (den + l + 1e-16)[:, None])


def _run_phase_d(num, den, ox, s):
    nhalf = GRID // NCORE

    return pl.pallas_call(
        _phase_d,
        grid=(NCORE, nhalf),
        in_specs=[
            pl.BlockSpec((BLK, D), lambda c, j: (c * nhalf + j, 0)),
            pl.BlockSpec((1, 1, NSUB, BLK), lambda c, j: (c, j, 0, 0)),
            pl.BlockSpec((BLK, D), lambda c, j: (c * nhalf + j, 0)),
            pl.BlockSpec((1, 2, BLK), lambda c, j: (c * nhalf + j, 0, 0)),
        ],
        out_specs=pl.BlockSpec((BLK, D), lambda c, j: (c * nhalf + j, 0)),
        out_shape=jax.ShapeDtypeStruct((N, D), jnp.float32),
    )(num, den, ox, s)


def kernel(x, edge_index, node_label, W_id, att):
    lbl3 = node_label.reshape(AGRID, 1, ABLK)
    att8 = jnp.zeros((8, D), jnp.float32).at[:2].set(att.reshape(2, D))
    ox, oxb, s = _run_phase_a(x, lbl3, W_id, att8)

    s2 = s.transpose(1, 0, 2).reshape(2, N)
    src = edge_index[0]
    dst = edge_index[1]
    zrows = jnp.zeros((ZROWS, D), jnp.float32)
    zden = jnp.zeros((DLOC,), jnp.float32)
    num, den = _run_phase_b(src, dst, s2[0], s2[1], oxb, zrows, zden)

    nhalf = GRID // NCORE
    den_t = den.reshape(NCORE, NSUB, nhalf, BLK).transpose(0, 2, 1, 3)
    s_d = s2.reshape(2, GRID, BLK).transpose(1, 0, 2)
    return _run_phase_d(num, den_t, ox, s_d)


# speedup vs baseline: 2.3250x; 2.3250x over previous
import jax
import jax.numpy as jnp
from jax import lax
from jax.experimental import pallas as pl
from jax.experimental.pallas import tpu as pltpu
from jax.experimental.pallas import tpu_sc as plsc

N = 10000
E = 160000
D = 256
BLK = 200
GRID = N // BLK
ABLK = 400
AGRID = N // ABLK
HALF = N // 2
NSUB = 16
NCORE = 2
CHUNK = E // NSUB
SEC = 2000
NSEC = CHUNK // SEC
SGROUPS = SEC // 16
G = 32
CAP = SEC + 2 * G
DLOC = HALF
ZROWS = 1000


def _phase_a(x_ref, lbl_ref, w_ref, att_ref, ox_ref, s_ref):
    xb = x_ref[...]
    lbl = lbl_ref[0, 0, :]
    acc = xb
    for i in range(7):
        m = (lbl == (i + 1)).astype(jnp.float32)[:, None]
        acc = acc + m * jnp.dot(xb, w_ref[i], preferred_element_type=jnp.float32)
    ox_ref[...] = acc
    s_ref[0, 0, :] = jnp.dot(acc, att_ref[0], preferred_element_type=jnp.float32)
    s_ref[0, 1, :] = jnp.dot(acc, att_ref[1], preferred_element_type=jnp.float32)


def _run_phase_a(x, lbl3, w_id, att8):
    return pl.pallas_call(
        _phase_a,
        grid=(AGRID,),
        in_specs=[
            pl.BlockSpec((ABLK, D), lambda j: (j, 0)),
            pl.BlockSpec((1, 1, ABLK), lambda j: (j, 0, 0)),
            pl.BlockSpec((7, D, D), lambda j: (0, 0, 0)),
            pl.BlockSpec((8, D), lambda j: (0, 0)),
        ],
        out_specs=[
            pl.BlockSpec((ABLK, D), lambda j: (j, 0)),
            pl.BlockSpec((1, 2, ABLK), lambda j: (j, 0, 0)),
        ],
        out_shape=[
            jax.ShapeDtypeStruct((N, D), jnp.float32),
            jax.ShapeDtypeStruct((AGRID, 2, ABLK), jnp.float32),
        ],
    )(x, lbl3, w_id, att8)


def _phase_b_impl(src_hbm, dst_hbm, sdst_hbm, ssrc_hbm, ox_hbm, zrows_hbm,
                  zden_hbm, num_hbm, den_hbm,
                  sdst_v, ssrc_v, rawsrc, rawdst, csrc, cdst, cw, rows_a,
                  rows_b, sc_idx, den_loc, num_sh, sem_a, sem_b):
    core = lax.axis_index("c")
    sub = lax.axis_index("s")
    base_dst = core * HALF

    @pl.when(sub < HALF // ZROWS)
    def _zero_num():
        pltpu.sync_copy(zrows_hbm, num_sh.at[pl.ds(sub * ZROWS, ZROWS)])

    pltpu.sync_copy(zden_hbm, den_loc)

    pltpu.sync_copy(sdst_hbm.at[pl.ds(base_dst, HALF)], sdst_v)
    pltpu.sync_copy(ssrc_hbm, ssrc_v)
    plsc.subcore_barrier()

    def start_gather(b, rows_k, sem_k):
        pltpu.async_copy(ox_hbm.at[csrc.at[pl.ds(b * G, G)]], rows_k, sem_k)

    def finish_block(b, rows_k, sem_k):
        pltpu.make_async_copy(ox_hbm.at[pl.ds(0, G)], rows_k, sem_k).wait()
        off = b * G
        for k in range(G // 16):
            sc_idx[pl.ds(16 * k, 16)] = cdst[pl.ds(off + 16 * k, 16)]

        def scale_group(q, c2):
            w16 = cw[pl.ds(off + 16 * q, 16)]
            for j in range(16):
                r = 16 * q + j
                wv = jnp.full((16,), w16[j], jnp.float32)
                for c in range(D // 16):
                    sl = pl.ds(16 * c, 16)
                    rows_k[r, sl] = rows_k[r, sl] * wv
            return c2

        lax.fori_loop(0, G // 16, scale_group, 0)
        pltpu.sync_copy(rows_k, num_sh.at[sc_idx], add=True)

    def do_section(sec, ptr):
        ebase = sub * CHUNK + sec * SEC
        pltpu.sync_copy(src_hbm.at[pl.ds(ebase, SEC)], rawsrc)
        pltpu.sync_copy(dst_hbm.at[pl.ds(ebase, SEC)], rawdst)

        def scan_group(g, p):
            src16 = rawsrc[pl.ds(g * 16, 16)]
            dst16 = rawdst[pl.ds(g * 16, 16)]
            local = dst16 - base_dst
            keep = ((local >= 0) & (local < HALF)) & (src16 != dst16)
            safe_local = jnp.where(keep, local, 0)
            a = (plsc.load_gather(sdst_v, [safe_local])
                 + plsc.load_gather(ssrc_v, [src16]))
            a = jnp.where(a > 0, a, 0.2 * a)
            w16 = jnp.exp(a)
            plsc.addupdate_scatter(den_loc, [safe_local], w16, mask=keep)
            csum = jnp.cumsum(keep.astype(jnp.int32))
            pos = p + csum - 1
            plsc.store_scatter(csrc, [pos], src16, mask=keep)
            plsc.store_scatter(cdst, [pos], safe_local, mask=keep)
            plsc.store_scatter(cw, [pos], w16, mask=keep)
            return p + jnp.max(csum)

        ptr = lax.fori_loop(0, SGROUPS, scan_group, ptr)

        nblk = ptr // G

        @pl.when(nblk > 0)
        def _prime_a():
            start_gather(0, rows_a, sem_a)

        @pl.when(nblk > 1)
        def _prime_b():
            start_gather(1, rows_b, sem_b)

        def outer(g2, c2):
            for k, (rk, sk) in enumerate(((rows_a, sem_a), (rows_b, sem_b))):
                b = g2 * 2 + k

                @pl.when(b < nblk)
                def _run():
                    finish_block(b, rk, sk)

                    @pl.when(b + 2 < nblk)
                    def _next():
                        start_gather(b + 2, rk, sk)
            return c2

        lax.fori_loop(0, (nblk + 1) // 2, outer, 0)
        rem = ptr - nblk * G
        for k in range(G // 16):
            sl = pl.ds(16 * k, 16)
            tmp_s = csrc[pl.ds(nblk * G + 16 * k, 16)]
            tmp_d = cdst[pl.ds(nblk * G + 16 * k, 16)]
            tmp_w = cw[pl.ds(nblk * G + 16 * k, 16)]
            csrc[sl] = tmp_s
            cdst[sl] = tmp_d
            cw[sl] = tmp_w
        return rem

    rem = lax.fori_loop(0, NSEC, do_section, jnp.int32(0))

    lane = lax.iota(jnp.int32, 16)
    for k in range(G // 16):
        pos = rem + lane + 16 * k
        plsc.store_scatter(csrc, [pos], jnp.zeros((16,), jnp.int32))
        plsc.store_scatter(cdst, [pos], jnp.zeros((16,), jnp.int32))
        plsc.store_scatter(cw, [pos], jnp.zeros((16,), jnp.float32))
    start_gather(0, rows_a, sem_a)
    finish_block(0, rows_a, sem_a)

    plsc.subcore_barrier()
    pltpu.sync_copy(den_loc, den_hbm.at[core, sub])

    @pl.when(sub < HALF // ZROWS)
    def _copy_out():
        row0 = sub * ZROWS
        pltpu.sync_copy(num_sh.at[pl.ds(row0, ZROWS)],
                        num_hbm.at[pl.ds(base_dst + row0, ZROWS)])


def _run_phase_b(src, dst, sdst, ssrc, ox, zrows, zden):
    mesh = plsc.VectorSubcoreMesh(core_axis_name="c", subcore_axis_name="s")
    kern = pl.kernel(
        _phase_b_impl,
        mesh=mesh,
        compiler_params=pltpu.CompilerParams(
            use_tc_tiling_on_sc=False, needs_layout_passes=False),
        out_type=[
            jax.ShapeDtypeStruct((N, D), jnp.float32),
            jax.ShapeDtypeStruct((NCORE, NSUB, DLOC), jnp.float32),
        ],
        scratch_types=[
            pltpu.VMEM((HALF,), jnp.float32),
            pltpu.VMEM((N,), jnp.float32),
            pltpu.VMEM((SEC,), jnp.int32),
            pltpu.VMEM((SEC,), jnp.int32),
            pltpu.VMEM((CAP,), jnp.int32),
            pltpu.VMEM((CAP,), jnp.int32),
            pltpu.VMEM((CAP,), jnp.float32),
            pltpu.VMEM((G, D), jnp.float32),
            pltpu.VMEM((G, D), jnp.float32),
            pltpu.VMEM((G,), jnp.int32),
            pltpu.VMEM((DLOC,), jnp.float32),
            pltpu.VMEM_SHARED((HALF, D), jnp.float32),
            pltpu.SemaphoreType.DMA,
            pltpu.SemaphoreType.DMA,
        ],
    )
    return kern(src, dst, sdst, ssrc, ox, zrows, zden)


def _phase_d(num_ref, den_ref, ox_ref, s_ref, out_ref):
    den = jnp.sum(den_ref[0, 0], axis=0)
    a = s_ref[0, 0, :] + s_ref[0, 1, :]
    a = jnp.where(a > 0, a, 0.2 * a)
    l = jnp.exp(a)
    oxb = ox_ref[...]
    out_ref[...] = ((num_ref[...] + l[:, None] * oxb)
                    / (den + l + 1e-16)[:, None])


def _run_phase_d(num, den, ox, s):
    nhalf = GRID // NCORE

    return pl.pallas_call(
        _phase_d,
        grid=(NCORE, nhalf),
        in_specs=[
            pl.BlockSpec((BLK, D), lambda c, j: (c * nhalf + j, 0)),
            pl.BlockSpec((1, 1, NSUB, BLK), lambda c, j: (c, j, 0, 0)),
            pl.BlockSpec((BLK, D), lambda c, j: (c * nhalf + j, 0)),
            pl.BlockSpec((1, 2, BLK), lambda c, j: (c * nhalf + j, 0, 0)),
        ],
        out_specs=pl.BlockSpec((BLK, D), lambda c, j: (c * nhalf + j, 0)),
        out_shape=jax.ShapeDtypeStruct((N, D), jnp.float32),
    )(num, den, ox, s)


def kernel(x, edge_index, node_label, W_id, att):
    lbl3 = node_label.reshape(AGRID, 1, ABLK)
    att8 = jnp.zeros((8, D), jnp.float32).at[:2].set(att.reshape(2, D))
    ox, s = _run_phase_a(x, lbl3, W_id, att8)

    s2 = s.transpose(1, 0, 2).reshape(2, N)
    src = edge_index[0]
    dst = edge_index[1]
    zrows = jnp.zeros((ZROWS, D), jnp.float32)
    zden = jnp.zeros((DLOC,), jnp.float32)
    num, den = _run_phase_b(src, dst, s2[0], s2[1], ox, zrows, zden)

    nhalf = GRID // NCORE
    den_t = den.reshape(NCORE, NSUB, nhalf, BLK).transpose(0, 2, 1, 3)
    s_d = s2.reshape(2, GRID, BLK).transpose(1, 0, 2)
    return _run_phase_d(num, den_t, ox, s_d)
